# baseline (speedup 1.0000x reference)
import functools

import jax
import jax.numpy as jnp
from jax import lax
from jax.experimental import pallas as pl
from jax.experimental.pallas import tpu as pltpu
from jax.experimental.pallas import tpu_sc as plsc

_BATCH = 16384
_EMBED = 64
_NC = 2
_NS = 16
_NW = _NC * _NS
_BPW = _BATCH // _NW
_FAN = 4
_SG = 136


def _plan(b):
    perm = jnp.argsort(b).astype(jnp.int32)
    bs = b[perm]
    qs = jnp.right_shift(bs, 7)
    ss = jnp.bitwise_and(bs, 127)
    jarr = jnp.arange(_BATCH, dtype=jnp.int32)
    prev = jnp.concatenate([qs[:1] - 1, qs[:-1]])
    nf = ((jarr % _BPW) == 0) | (qs != prev)
    rank = jnp.cumsum(nf.astype(jnp.int32)) - 1
    rank_w = rank - jnp.repeat(rank[::_BPW], _BPW)
    counts = rank_w[_BPW - 1::_BPW] + 1
    ngroups = (counts + _FAN - 1) // _FAN
    bl = jnp.zeros((_NW, _BPW), jnp.int32).at[jarr // _BPW, rank_w].set(qs)
    sg = jax.vmap(
        lambda r: jnp.searchsorted(r, _FAN * jnp.arange(_SG), side="left")
    )(rank_w.reshape(_NW, _BPW)).astype(jnp.int32)
    slot = jnp.bitwise_and(rank_w, _FAN - 1)
    return (bl.reshape(-1), sg.reshape(-1), ss, slot, perm,
            ngroups.astype(jnp.int32))


def _scalar(ref, j):
    return jnp.max(plsc.load_gather(ref, [jnp.full((16,), j, jnp.int32)]))


def _gather_body(ubl_hbm, usg_hbm, uss_hbm, uslot_hbm, upos_hbm, ung_hbm,
                 ibl_hbm, isg_hbm, iss_hbm, islot_hbm, ipos_hbm, ing_hbm,
                 utabT_hbm, itabT_hbm, uout_hbm, iout_hbm,
                 bl_v, sg_v, s_v, slot_v, pos_v, ng_v, slab_v, out_v,
                 sem_a, sem_b):
    wid = lax.axis_index("s") * _NC + lax.axis_index("c")
    base = wid * _BPW
    iota16 = lax.iota(jnp.int32, 16)

    for bl_hbm, sg_hbm, ss_hbm, slot_hbm, pos_hbm, ng_hbm, tab_hbm, out_hbm in (
            (ubl_hbm, usg_hbm, uss_hbm, uslot_hbm, upos_hbm, ung_hbm,
             utabT_hbm, uout_hbm),
            (ibl_hbm, isg_hbm, iss_hbm, islot_hbm, ipos_hbm, ing_hbm,
             itabT_hbm, iout_hbm)):
        pltpu.sync_copy(bl_hbm.at[pl.ds(base, _BPW)], bl_v)
        pltpu.sync_copy(sg_hbm.at[pl.ds(wid * _SG, _SG)], sg_v)
        pltpu.sync_copy(ss_hbm.at[pl.ds(base, _BPW)], s_v)
        pltpu.sync_copy(slot_hbm.at[pl.ds(base, _BPW)], slot_v)
        pltpu.sync_copy(pos_hbm.at[pl.ds(base, _BPW)], pos_v)
        pltpu.sync_copy(ng_hbm, ng_v)
        ng = _scalar(ng_v, wid)

        def group(g, carry):
            for k in range(_FAN):
                q = _scalar(bl_v, g * _FAN + k)
                pltpu.async_copy(
                    tab_hbm.at[:, pl.ds(pl.multiple_of(q * 128, 128), 128)],
                    slab_v.at[k], sem_a)
            for k in range(_FAN):
                pltpu.make_async_copy(tab_hbm.at[:, pl.ds(0, 128)],
                                      slab_v.at[k], sem_a).wait()
            j0 = _scalar(sg_v, g)
            j1 = _scalar(sg_v, g + 1)

            def ext(j, c2):
                sb = plsc.load_gather(s_v, [jnp.full((16,), j, jnp.int32)])
                kb = plsc.load_gather(slot_v, [jnp.full((16,), j, jnp.int32)])
                for gg in range(4):
                    val = plsc.load_gather(slab_v, [kb, iota16 + 16 * gg, sb])
                    out_v[pl.ds(j * _EMBED + 16 * gg, 16)] = val
                return c2

            lax.fori_loop(j0, j1, ext, 0)
            return carry

        lax.fori_loop(0, ng, group, 0)

        def scat(j, c2):
            p = _scalar(pos_v, j)
            pltpu.async_copy(
                out_v.at[pl.ds(j * _EMBED, _EMBED)],
                out_hbm.at[pl.ds(pl.multiple_of(p * _EMBED, _EMBED), _EMBED)],
                sem_b)
            return c2

        lax.fori_loop(0, _BPW, scat, 0)
        pltpu.make_async_copy(out_hbm.at[pl.ds(0, _BPW * _EMBED)], out_v,
                              sem_b).wait()


_gather = functools.partial(
    pl.kernel,
    mesh=plsc.VectorSubcoreMesh(core_axis_name="c", subcore_axis_name="s"),
    out_type=(
        jax.ShapeDtypeStruct((_BATCH * _EMBED,), jnp.float32),
        jax.ShapeDtypeStruct((_BATCH * _EMBED,), jnp.float32),
    ),
    scratch_types=[
        pltpu.VMEM((_BPW,), jnp.int32),
        pltpu.VMEM((_SG,), jnp.int32),
        pltpu.VMEM((_BPW,), jnp.int32),
        pltpu.VMEM((_BPW,), jnp.int32),
        pltpu.VMEM((_BPW,), jnp.int32),
        pltpu.VMEM((_NW,), jnp.int32),
        pltpu.VMEM((_FAN, _EMBED, 128), jnp.float32),
        pltpu.VMEM((_BPW * _EMBED,), jnp.float32),
        pltpu.SemaphoreType.DMA,
        pltpu.SemaphoreType.DMA,
    ],
    compiler_params=pltpu.CompilerParams(needs_layout_passes=False),
)(_gather_body)


_TILE = 1024


def _mlp_body(ue_ref, ie_ref, w1u_ref, w1i_ref, b1_ref, w2_ref, b2_ref,
              w3_ref, b3_ref, w4_ref, b4_ref, out_ref):
    dot = functools.partial(jnp.dot, preferred_element_type=jnp.float32)
    bf = jnp.bfloat16
    h = dot(ue_ref[...].astype(bf), w1u_ref[...]) + dot(
        ie_ref[...].astype(bf), w1i_ref[...])
    h = jnp.maximum(h + b1_ref[...], 0.0)
    h = jnp.maximum(dot(h.astype(bf), w2_ref[...]) + b2_ref[...], 0.0)
    h = jnp.maximum(dot(h.astype(bf), w3_ref[...]) + b3_ref[...], 0.0)
    out_ref[...] = jnp.maximum(dot(h.astype(bf), w4_ref[...]) + b4_ref[...], 0.0)


def _mlp(ue, ie, W1u, W1i, b1, W2, b2, W3, b3, W4, b4):
    full = lambda r, c: pl.BlockSpec((r, c), lambda i: (0, 0))
    return pl.pallas_call(
        _mlp_body,
        grid=(_BATCH // _TILE,),
        in_specs=[
            pl.BlockSpec((_TILE, _EMBED), lambda i: (i, 0)),
            pl.BlockSpec((_TILE, _EMBED), lambda i: (i, 0)),
            full(_EMBED, 1024), full(_EMBED, 1024), full(1, 1024),
            full(1024, 512), full(1, 512),
            full(512, 256), full(1, 256),
            full(256, 128), full(1, 128),
        ],
        out_specs=pl.BlockSpec((_TILE, 128), lambda i: (i, 0)),
        out_shape=jax.ShapeDtypeStruct((_BATCH, 128), jnp.float32),
    )(ue, ie, W1u, W1i, b1, W2, b2, W3, b3, W4, b4)


def kernel(user_batch, item_batch, user_table, item_table,
           W1, b1, W2, b2, W3, b3, W4, b4):
    uplan = _plan(user_batch.astype(jnp.int32))
    iplan = _plan(item_batch.astype(jnp.int32))
    uef, ief = _gather(*uplan, *iplan, user_table.T, item_table.T)
    bf = jnp.bfloat16
    return _mlp(uef.reshape(_BATCH, _EMBED), ief.reshape(_BATCH, _EMBED),
                W1[:_EMBED].astype(bf), W1[_EMBED:].astype(bf),
                b1.reshape(1, -1), W2.astype(bf), b2.reshape(1, -1),
                W3.astype(bf), b3.reshape(1, -1), W4.astype(bf),
                b4.reshape(1, -1))


# speedup vs baseline: 1.0599x; 1.0599x over previous
import functools

import jax
import jax.numpy as jnp
from jax import lax
from jax.experimental import pallas as pl
from jax.experimental.pallas import tpu as pltpu
from jax.experimental.pallas import tpu_sc as plsc

_BATCH = 16384
_EMBED = 64
_NC = 2
_NS = 16
_NW = _NC * _NS
_BPW = _BATCH // _NW
_FAN = 4


def _gather_body(uq_hbm, iq_hbm, us_hbm, is_hbm, utabT_hbm, itabT_hbm,
                 uout_hbm, iout_hbm, q_v, s_v, slab_v, out_v, sem_a, sem_b):
    wid = lax.axis_index("s") * _NC + lax.axis_index("c")
    base = wid * _BPW
    iota16 = lax.iota(jnp.int32, 16)
    nchunk = _BPW // _FAN

    for q_hbm, sl_hbm, tab_hbm, out_hbm in (
            (uq_hbm, us_hbm, utabT_hbm, uout_hbm),
            (iq_hbm, is_hbm, itabT_hbm, iout_hbm)):
        pltpu.sync_copy(q_hbm.at[pl.ds(base, _BPW)], q_v)
        pltpu.sync_copy(sl_hbm.at[pl.ds(base, _BPW)], s_v)

        def fire(c, buf, sem):
            for k in range(_FAN):
                j = c * _FAN + k
                q = jnp.max(plsc.load_gather(
                    q_v, [jnp.full((16,), j, jnp.int32)]))
                pltpu.async_copy(
                    tab_hbm.at[:, pl.ds(pl.multiple_of(q * 128, 128), 128)],
                    slab_v.at[buf * _FAN + k], sem)

        def drain(sem, buf):
            for k in range(_FAN):
                pltpu.make_async_copy(tab_hbm.at[:, pl.ds(0, 128)],
                                      slab_v.at[buf * _FAN + k], sem).wait()

        def extract(c, buf):
            for k in range(_FAN):
                j = c * _FAN + k
                sb = plsc.load_gather(s_v, [jnp.full((16,), j, jnp.int32)])
                kb = jnp.full((16,), buf * _FAN + k, jnp.int32)
                for g in range(4):
                    val = plsc.load_gather(slab_v, [kb, iota16 + 16 * g, sb])
                    out_v[pl.ds(j * _EMBED + 16 * g, 16)] = val

        def pair(g, carry):
            c0 = 2 * g
            c1 = c0 + 1
            fire(c0, 0, sem_a)
            drain(sem_a, 0)
            extract(c0, 0)
            fire(c1, 1, sem_b)
            drain(sem_b, 1)
            extract(c1, 1)
            return carry

        lax.fori_loop(0, nchunk // 2, pair, 0)
        pltpu.sync_copy(out_v, out_hbm.at[pl.ds(base * _EMBED, _BPW * _EMBED)])


_gather = functools.partial(
    pl.kernel,
    mesh=plsc.VectorSubcoreMesh(core_axis_name="c", subcore_axis_name="s"),
    out_type=(
        jax.ShapeDtypeStruct((_BATCH * _EMBED,), jnp.float32),
        jax.ShapeDtypeStruct((_BATCH * _EMBED,), jnp.float32),
    ),
    scratch_types=[
        pltpu.VMEM((_BPW,), jnp.int32),
        pltpu.VMEM((_BPW,), jnp.int32),
        pltpu.VMEM((2 * _FAN, _EMBED, 128), jnp.float32),
        pltpu.VMEM((_BPW * _EMBED,), jnp.float32),
        pltpu.SemaphoreType.DMA,
        pltpu.SemaphoreType.DMA,
    ],
    compiler_params=pltpu.CompilerParams(needs_layout_passes=False),
)(_gather_body)


_TILE = 1024


def _mlp_body(ue_ref, ie_ref, w1u_ref, w1i_ref, b1_ref, w2_ref, b2_ref,
              w3_ref, b3_ref, w4_ref, b4_ref, out_ref):
    dot = functools.partial(jnp.dot, preferred_element_type=jnp.float32)
    bf = jnp.bfloat16
    h = dot(ue_ref[...].astype(bf), w1u_ref[...]) + dot(
        ie_ref[...].astype(bf), w1i_ref[...])
    h = jnp.maximum(h + b1_ref[...], 0.0)
    h = jnp.maximum(dot(h.astype(bf), w2_ref[...]) + b2_ref[...], 0.0)
    h = jnp.maximum(dot(h.astype(bf), w3_ref[...]) + b3_ref[...], 0.0)
    out_ref[...] = jnp.maximum(dot(h.astype(bf), w4_ref[...]) + b4_ref[...], 0.0)


def _mlp(ue, ie, W1u, W1i, b1, W2, b2, W3, b3, W4, b4):
    full = lambda r, c: pl.BlockSpec((r, c), lambda i: (0, 0))
    return pl.pallas_call(
        _mlp_body,
        grid=(_BATCH // _TILE,),
        in_specs=[
            pl.BlockSpec((_TILE, _EMBED), lambda i: (i, 0)),
            pl.BlockSpec((_TILE, _EMBED), lambda i: (i, 0)),
            full(_EMBED, 1024), full(_EMBED, 1024), full(1, 1024),
            full(1024, 512), full(1, 512),
            full(512, 256), full(1, 256),
            full(256, 128), full(1, 128),
        ],
        out_specs=pl.BlockSpec((_TILE, 128), lambda i: (i, 0)),
        out_shape=jax.ShapeDtypeStruct((_BATCH, 128), jnp.float32),
    )(ue, ie, W1u, W1i, b1, W2, b2, W3, b3, W4, b4)


def kernel(user_batch, item_batch, user_table, item_table,
           W1, b1, W2, b2, W3, b3, W4, b4):
    ub = user_batch.astype(jnp.int32)
    ib = item_batch.astype(jnp.int32)
    uef, ief = _gather(
        jnp.right_shift(ub, 7), jnp.right_shift(ib, 7),
        jnp.bitwise_and(ub, 127), jnp.bitwise_and(ib, 127),
        user_table.T, item_table.T)
    bf = jnp.bfloat16
    return _mlp(uef.reshape(_BATCH, _EMBED), ief.reshape(_BATCH, _EMBED),
                W1[:_EMBED].astype(bf), W1[_EMBED:].astype(bf),
                b1.reshape(1, -1), W2.astype(bf), b2.reshape(1, -1),
                W3.astype(bf), b3.reshape(1, -1), W4.astype(bf),
                b4.reshape(1, -1))


# speedup vs baseline: 1.2402x; 1.1702x over previous
import functools

import jax
import jax.numpy as jnp
from jax import lax
from jax.experimental import pallas as pl
from jax.experimental.pallas import tpu as pltpu
from jax.experimental.pallas import tpu_sc as plsc

_BATCH = 16384
_EMBED = 64
_NC = 2
_NS = 16
_NW = _NC * _NS
_BPW = _BATCH // _NW
_FAN = 8


def _gather_body(uq_hbm, iq_hbm, us_hbm, is_hbm, utabT_hbm, itabT_hbm,
                 uout_hbm, iout_hbm, q_v, s_v, slab_v, out_v, sem_a, sem_b):
    wid = lax.axis_index("s") * _NC + lax.axis_index("c")
    base = wid * _BPW
    iota16 = lax.iota(jnp.int32, 16)
    nchunk = _BPW // _FAN

    for q_hbm, sl_hbm, tab_hbm, out_hbm in (
            (uq_hbm, us_hbm, utabT_hbm, uout_hbm),
            (iq_hbm, is_hbm, itabT_hbm, iout_hbm)):
        pltpu.sync_copy(q_hbm.at[pl.ds(base, _BPW)], q_v)
        pltpu.sync_copy(sl_hbm.at[pl.ds(base, _BPW)], s_v)

        def fire(c, buf, sem):
            for k in range(_FAN):
                j = c * _FAN + k
                q = jnp.max(plsc.load_gather(
                    q_v, [jnp.full((16,), j, jnp.int32)]))
                pltpu.async_copy(
                    tab_hbm.at[:, pl.ds(pl.multiple_of(q * 128, 128), 128)],
                    slab_v.at[buf * _FAN + k], sem)

        def drain(sem, buf):
            for k in range(_FAN):
                pltpu.make_async_copy(tab_hbm.at[:, pl.ds(0, 128)],
                                      slab_v.at[buf * _FAN + k], sem).wait()

        def extract(c, buf):
            for k in range(_FAN):
                j = c * _FAN + k
                sb = plsc.load_gather(s_v, [jnp.full((16,), j, jnp.int32)])
                kb = jnp.full((16,), buf * _FAN + k, jnp.int32)
                for g in range(4):
                    val = plsc.load_gather(slab_v, [kb, iota16 + 16 * g, sb])
                    out_v[pl.ds(j * _EMBED + 16 * g, 16)] = val

        def chunk(c, carry):
            fire(c, 0, sem_a)
            drain(sem_a, 0)
            extract(c, 0)
            return carry

        lax.fori_loop(0, nchunk, chunk, 0)
        pltpu.sync_copy(out_v, out_hbm.at[pl.ds(base * _EMBED, _BPW * _EMBED)])


_gather = functools.partial(
    pl.kernel,
    mesh=plsc.VectorSubcoreMesh(core_axis_name="c", subcore_axis_name="s"),
    out_type=(
        jax.ShapeDtypeStruct((_BATCH * _EMBED,), jnp.float32),
        jax.ShapeDtypeStruct((_BATCH * _EMBED,), jnp.float32),
    ),
    scratch_types=[
        pltpu.VMEM((_BPW,), jnp.int32),
        pltpu.VMEM((_BPW,), jnp.int32),
        pltpu.VMEM((_FAN, _EMBED, 128), jnp.float32),
        pltpu.VMEM((_BPW * _EMBED,), jnp.float32),
        pltpu.SemaphoreType.DMA,
        pltpu.SemaphoreType.DMA,
    ],
    compiler_params=pltpu.CompilerParams(needs_layout_passes=False),
)(_gather_body)


_TILE = 1024


def _mlp_body(ue_ref, ie_ref, w1u_ref, w1i_ref, b1_ref, w2_ref, b2_ref,
              w3_ref, b3_ref, w4_ref, b4_ref, out_ref):
    dot = functools.partial(jnp.dot, preferred_element_type=jnp.float32)
    bf = jnp.bfloat16
    h = dot(ue_ref[...].astype(bf), w1u_ref[...]) + dot(
        ie_ref[...].astype(bf), w1i_ref[...])
    h = jnp.maximum(h + b1_ref[...], 0.0)
    h = jnp.maximum(dot(h.astype(bf), w2_ref[...]) + b2_ref[...], 0.0)
    h = jnp.maximum(dot(h.astype(bf), w3_ref[...]) + b3_ref[...], 0.0)
    out_ref[...] = jnp.maximum(dot(h.astype(bf), w4_ref[...]) + b4_ref[...], 0.0)


def _mlp(ue, ie, W1u, W1i, b1, W2, b2, W3, b3, W4, b4):
    full = lambda r, c: pl.BlockSpec((r, c), lambda i: (0, 0))
    return pl.pallas_call(
        _mlp_body,
        grid=(_BATCH // _TILE,),
        in_specs=[
            pl.BlockSpec((_TILE, _EMBED), lambda i: (i, 0)),
            pl.BlockSpec((_TILE, _EMBED), lambda i: (i, 0)),
            full(_EMBED, 1024), full(_EMBED, 1024), full(1, 1024),
            full(1024, 512), full(1, 512),
            full(512, 256), full(1, 256),
            full(256, 128), full(1, 128),
        ],
        out_specs=pl.BlockSpec((_TILE, 128), lambda i: (i, 0)),
        out_shape=jax.ShapeDtypeStruct((_BATCH, 128), jnp.float32),
    )(ue, ie, W1u, W1i, b1, W2, b2, W3, b3, W4, b4)


def kernel(user_batch, item_batch, user_table, item_table,
           W1, b1, W2, b2, W3, b3, W4, b4):
    ub = user_batch.astype(jnp.int32)
    ib = item_batch.astype(jnp.int32)
    uef, ief = _gather(
        jnp.right_shift(ub, 7), jnp.right_shift(ib, 7),
        jnp.bitwise_and(ub, 127), jnp.bitwise_and(ib, 127),
        user_table.T, item_table.T)
    bf = jnp.bfloat16
    return _mlp(uef.reshape(_BATCH, _EMBED), ief.reshape(_BATCH, _EMBED),
                W1[:_EMBED].astype(bf), W1[_EMBED:].astype(bf),
                b1.reshape(1, -1), W2.astype(bf), b2.reshape(1, -1),
                W3.astype(bf), b3.reshape(1, -1), W4.astype(bf),
                b4.reshape(1, -1))


# speedup vs baseline: 1.3551x; 1.0926x over previous
import functools

import jax
import jax.numpy as jnp
from jax import lax
from jax.experimental import pallas as pl
from jax.experimental.pallas import tpu as pltpu
from jax.experimental.pallas import tpu_sc as plsc

_BATCH = 16384
_EMBED = 64
_NC = 2
_NS = 16
_NW = _NC * _NS
_BPW = _BATCH // _NW
_FAN = 8


def _gather_body(uq_hbm, iq_hbm, us_hbm, is_hbm, utabT_hbm, itabT_hbm,
                 uout_hbm, iout_hbm, q_v, s_v, slab_v, out_v, sem_a, sem_b):
    wid = lax.axis_index("s") * _NC + lax.axis_index("c")
    base = wid * _BPW
    iota16 = lax.iota(jnp.int32, 16)
    nchunk = _BPW // _FAN

    for q_hbm, sl_hbm, tab_hbm, out_hbm in (
            (uq_hbm, us_hbm, utabT_hbm, uout_hbm),
            (iq_hbm, is_hbm, itabT_hbm, iout_hbm)):
        pltpu.sync_copy(q_hbm.at[pl.ds(base, _BPW)], q_v)
        pltpu.sync_copy(sl_hbm.at[pl.ds(base, _BPW)], s_v)

        def fire(c, buf, sem):
            for k in range(_FAN):
                j = c * _FAN + k
                q = jnp.max(plsc.load_gather(
                    q_v, [jnp.full((16,), j, jnp.int32)]))
                pltpu.async_copy(
                    tab_hbm.at[:, pl.ds(pl.multiple_of(q * 128, 128), 128)],
                    slab_v.at[buf * _FAN + k], sem)

        def drain(sem, buf):
            for k in range(_FAN):
                pltpu.make_async_copy(tab_hbm.at[:, pl.ds(0, 128)],
                                      slab_v.at[buf * _FAN + k], sem).wait()

        def extract(c, buf):
            for k in range(_FAN):
                j = c * _FAN + k
                sb = plsc.load_gather(s_v, [jnp.full((16,), j, jnp.int32)])
                kb = jnp.full((16,), buf * _FAN + k, jnp.int32)
                for g in range(4):
                    val = plsc.load_gather(slab_v, [kb, iota16 + 16 * g, sb])
                    out_v[pl.ds(j * _EMBED + 16 * g, 16)] = val

        def chunk(c, carry):
            fire(c, 0, sem_a)
            for k in range(_FAN):
                pltpu.make_async_copy(tab_hbm.at[:, pl.ds(0, 128)],
                                      slab_v.at[k], sem_a).wait()
                j = c * _FAN + k
                sb = plsc.load_gather(s_v, [jnp.full((16,), j, jnp.int32)])
                kb = jnp.full((16,), k, jnp.int32)
                for g in range(4):
                    val = plsc.load_gather(slab_v, [kb, iota16 + 16 * g, sb])
                    out_v[pl.ds(j * _EMBED + 16 * g, 16)] = val
            return carry

        lax.fori_loop(0, nchunk, chunk, 0)
        pltpu.sync_copy(out_v, out_hbm.at[pl.ds(base * _EMBED, _BPW * _EMBED)])


_gather = functools.partial(
    pl.kernel,
    mesh=plsc.VectorSubcoreMesh(core_axis_name="c", subcore_axis_name="s"),
    out_type=(
        jax.ShapeDtypeStruct((_BATCH * _EMBED,), jnp.float32),
        jax.ShapeDtypeStruct((_BATCH * _EMBED,), jnp.float32),
    ),
    scratch_types=[
        pltpu.VMEM((_BPW,), jnp.int32),
        pltpu.VMEM((_BPW,), jnp.int32),
        pltpu.VMEM((_FAN, _EMBED, 128), jnp.float32),
        pltpu.VMEM((_BPW * _EMBED,), jnp.float32),
        pltpu.SemaphoreType.DMA,
        pltpu.SemaphoreType.DMA,
    ],
    compiler_params=pltpu.CompilerParams(needs_layout_passes=False),
)(_gather_body)


_TILE = 1024


def _mlp_body(ue_ref, ie_ref, w1u_ref, w1i_ref, b1_ref, w2_ref, b2_ref,
              w3_ref, b3_ref, w4_ref, b4_ref, out_ref):
    dot = functools.partial(jnp.dot, preferred_element_type=jnp.float32)
    bf = jnp.bfloat16
    h = dot(ue_ref[...].astype(bf), w1u_ref[...]) + dot(
        ie_ref[...].astype(bf), w1i_ref[...])
    h = jnp.maximum(h + b1_ref[...], 0.0)
    h = jnp.maximum(dot(h.astype(bf), w2_ref[...]) + b2_ref[...], 0.0)
    h = jnp.maximum(dot(h.astype(bf), w3_ref[...]) + b3_ref[...], 0.0)
    out_ref[...] = jnp.maximum(dot(h.astype(bf), w4_ref[...]) + b4_ref[...], 0.0)


def _mlp(ue, ie, W1u, W1i, b1, W2, b2, W3, b3, W4, b4):
    full = lambda r, c: pl.BlockSpec((r, c), lambda i: (0, 0))
    return pl.pallas_call(
        _mlp_body,
        grid=(_BATCH // _TILE,),
        in_specs=[
            pl.BlockSpec((_TILE, _EMBED), lambda i: (i, 0)),
            pl.BlockSpec((_TILE, _EMBED), lambda i: (i, 0)),
            full(_EMBED, 1024), full(_EMBED, 1024), full(1, 1024),
            full(1024, 512), full(1, 512),
            full(512, 256), full(1, 256),
            full(256, 128), full(1, 128),
        ],
        out_specs=pl.BlockSpec((_TILE, 128), lambda i: (i, 0)),
        out_shape=jax.ShapeDtypeStruct((_BATCH, 128), jnp.float32),
    )(ue, ie, W1u, W1i, b1, W2, b2, W3, b3, W4, b4)


def kernel(user_batch, item_batch, user_table, item_table,
           W1, b1, W2, b2, W3, b3, W4, b4):
    ub = user_batch.astype(jnp.int32)
    ib = item_batch.astype(jnp.int32)
    uef, ief = _gather(
        jnp.right_shift(ub, 7), jnp.right_shift(ib, 7),
        jnp.bitwise_and(ub, 127), jnp.bitwise_and(ib, 127),
        user_table.T, item_table.T)
    bf = jnp.bfloat16
    return _mlp(uef.reshape(_BATCH, _EMBED), ief.reshape(_BATCH, _EMBED),
                W1[:_EMBED].astype(bf), W1[_EMBED:].astype(bf),
                b1.reshape(1, -1), W2.astype(bf), b2.reshape(1, -1),
                W3.astype(bf), b3.reshape(1, -1), W4.astype(bf),
                b4.reshape(1, -1))
